# Initial kernel scaffold; baseline (speedup 1.0000x reference)
import jax
import jax.numpy as jnp
from jax.experimental import pallas as pl


def kernel(logits, generated, top_k):
    raise NotImplementedError("write your pallas kernel here")


# speedup vs baseline: 21.8315x; 21.8315x over previous
import functools

import jax
import jax.numpy as jnp
from jax import lax
from jax.experimental import pallas as pl
from jax.experimental.pallas import tpu as pltpu
from jax.experimental.pallas import tpu_sc as plsc

_B, _V = 128, 100000
_TEMPERATURE = 0.8
_TOP_P = 0.9
_REP_PENALTY = 1.2

def _gumbel():
    return jax.random.gumbel(jax.random.key(42), (_B, _V), dtype=jnp.float32)


_NC, _NS, _L = 2, 16, 16
_NW = _NC * _NS
_ROWS_PER_W = _B // _NW
_ELEMS_PER_W = _ROWS_PER_W * _V
_IDX_PER_W = _ROWS_PER_W * 512
_IDX_ROWS = _IDX_PER_W // 128


def _sc_penalty_body(log_hbm, fidx_hbm, out_hbm, idx_v, vals_v, sem):
    wid = lax.axis_index("s") * _NC + lax.axis_index("c")
    base = wid * _ELEMS_PER_W
    pltpu.sync_copy(log_hbm.at[pl.ds(base, _ELEMS_PER_W)],
                    out_hbm.at[pl.ds(base, _ELEMS_PER_W)])
    pltpu.sync_copy(fidx_hbm.at[wid], idx_v)
    descs = [pltpu.async_copy(log_hbm.at[idx_v.at[j]], vals_v.at[j], sem)
             for j in range(_IDX_ROWS)]
    for d in descs:
        d.wait()
    for j in range(_IDX_ROWS):
        for i in range(128 // _L):
            v = vals_v[j, pl.ds(i * _L, _L)]
            vals_v[j, pl.ds(i * _L, _L)] = jnp.where(
                v > 0.0, v / _REP_PENALTY, v * _REP_PENALTY)
    descs = [pltpu.async_copy(vals_v.at[j], out_hbm.at[idx_v.at[j]], sem)
             for j in range(_IDX_ROWS)]
    for d in descs:
        d.wait()


@functools.cache
def _sc_penalty():
    return pl.kernel(
        _sc_penalty_body,
        mesh=plsc.VectorSubcoreMesh(
            core_axis_name="c", subcore_axis_name="s", num_cores=_NC),
        out_type=jax.ShapeDtypeStruct((_B * _V,), jnp.float32),
        scratch_types=[
            pltpu.VMEM((_IDX_ROWS, 128), jnp.int32),
            pltpu.VMEM((_IDX_ROWS, 128), jnp.float32),
            pltpu.SemaphoreType.DMA,
        ],
    )


_ROWS_PER_BLK = 8
_GRID = _B // _ROWS_PER_BLK


def _avg_ceil(lo, hi):
    fl = (lo >> 1) + (hi >> 1) + (lo & hi & 1)
    return fl + ((lo ^ hi) & 1)


def _tc_body(kk_ref, x_ref, g_ref, probs_ref, tok_ref, s_ref, e_ref):
    x = x_ref[...] / jnp.float32(_TEMPERATURE)
    b = lax.bitcast_convert_type(x, jnp.int32)
    s = b ^ ((b >> 31) & jnp.int32(0x7FFFFFFF))
    s_ref[...] = s
    smax = jnp.max(s, axis=1, keepdims=True)
    smin = jnp.min(s, axis=1, keepdims=True)
    kk = kk_ref[0]

    def cond1(c):
        return jnp.any(c[0] < c[1])

    def body1(c):
        lo, hi = c
        mid = _avg_ceil(lo, hi)
        cnt = jnp.sum((s_ref[...] >= mid).astype(jnp.int32), axis=1,
                      keepdims=True)
        ge = cnt >= kk
        return (jnp.where(ge, mid, lo), jnp.where(ge, hi, mid - 1))

    tstar, _ = lax.while_loop(cond1, body1, (smin, smax))

    m = jnp.max(x, axis=1, keepdims=True)
    e = jnp.where(s >= tstar, jnp.exp(x - m), jnp.float32(0.0))
    e_ref[...] = e
    z = jnp.sum(e, axis=1, keepdims=True)
    target = z * jnp.float32(_TOP_P)

    def body2(c):
        lo, hi = c
        mid = _avg_ceil(lo, hi)
        mass = jnp.sum(jnp.where(s_ref[...] >= mid, e_ref[...],
                                 jnp.float32(0.0)), axis=1, keepdims=True)
        ge = mass >= target
        return (jnp.where(ge, mid, lo), jnp.where(ge, hi, mid - 1))

    t0, _ = lax.while_loop(cond1, body2, (tstar, smax))

    at_t0 = s_ref[...] == t0
    mass_above = jnp.sum(jnp.where(s_ref[...] > t0, e_ref[...],
                                   jnp.float32(0.0)), axis=1, keepdims=True)
    e_tie = jnp.max(jnp.where(at_t0, e_ref[...], jnp.float32(0.0)),
                    axis=1, keepdims=True)
    n_keep = jnp.ceil((target - mass_above) / e_tie).astype(jnp.int32)

    def body3(c):
        lo, hi = c
        mid = (lo >> 1) + (hi >> 1) + (lo & hi & 1)
        ii = lax.broadcasted_iota(jnp.int32, at_t0.shape, 1)
        cnt = jnp.sum((at_t0 & (ii < mid)).astype(jnp.int32), axis=1,
                      keepdims=True)
        ge = cnt >= n_keep
        return (jnp.where(ge, lo, mid + 1), jnp.where(ge, mid, hi))

    zero = jnp.zeros_like(t0)
    cidx, _ = lax.while_loop(cond1, body3, (zero, zero + jnp.int32(_V)))

    ii = lax.broadcasted_iota(jnp.int32, at_t0.shape, 1)
    keep = (s > t0) | (at_t0 & (ii < cidx))
    ekeep = jnp.where(keep, e, jnp.float32(0.0))
    zs = jnp.sum(ekeep, axis=1, keepdims=True)
    p = ekeep / zs
    probs_ref[...] = p

    score = jnp.log(p + jnp.float32(1e-20)) + g_ref[...]
    mx = jnp.max(score, axis=1, keepdims=True)
    ii = lax.broadcasted_iota(jnp.int32, score.shape, 1)
    idx = jnp.min(jnp.where(score >= mx, ii, jnp.int32(_V)), axis=1)
    tok_ref[...] = idx[:, None]


def _tc_sample(plog, gumbel, kk):
    grid_spec = pltpu.PrefetchScalarGridSpec(
        num_scalar_prefetch=1,
        grid=(_GRID,),
        in_specs=[
            pl.BlockSpec((_ROWS_PER_BLK, _V), lambda i, kk: (i, 0)),
            pl.BlockSpec((_ROWS_PER_BLK, _V), lambda i, kk: (i, 0)),
        ],
        out_specs=[
            pl.BlockSpec((_ROWS_PER_BLK, _V), lambda i, kk: (i, 0)),
            pl.BlockSpec((_ROWS_PER_BLK, 1), lambda i, kk: (i, 0)),
        ],
        scratch_shapes=[
            pltpu.VMEM((_ROWS_PER_BLK, _V), jnp.int32),
            pltpu.VMEM((_ROWS_PER_BLK, _V), jnp.float32),
        ],
    )
    return pl.pallas_call(
        _tc_body,
        grid_spec=grid_spec,
        out_shape=[
            jax.ShapeDtypeStruct((_B, _V), jnp.float32),
            jax.ShapeDtypeStruct((_B, 1), jnp.int32),
        ],
    )(kk, plog, gumbel)


def kernel(logits, generated, top_k):
    fidx = (generated.astype(jnp.int32)
            + jnp.arange(_B, dtype=jnp.int32)[:, None] * _V)
    fidx = fidx.reshape(_NW, _IDX_ROWS, 128)
    plog = _sc_penalty()(logits.reshape(-1), fidx).reshape(_B, _V)
    kk = jnp.minimum(jnp.asarray(top_k, jnp.int32), _V).reshape(1)
    probs, tok = _tc_sample(plog, _gumbel(), kk)
    return probs, tok.reshape(_B)


# speedup vs baseline: 24.2963x; 1.1129x over previous
import functools

import jax
import jax.numpy as jnp
from jax import lax
from jax.experimental import pallas as pl
from jax.experimental.pallas import tpu as pltpu
from jax.experimental.pallas import tpu_sc as plsc

_B, _V = 128, 100000
_TEMPERATURE = 0.8
_TOP_P = 0.9
_REP_PENALTY = 1.2

def _gumbel():
    return jax.random.gumbel(jax.random.key(42), (_B, _V), dtype=jnp.float32)


_NC, _NS, _L = 2, 16, 16
_NW = _NC * _NS
_ROWS_PER_W = _B // _NW
_ELEMS_PER_W = _ROWS_PER_W * _V
_IDX_PER_W = _ROWS_PER_W * 512
_IDX_ROWS = _IDX_PER_W // 128


def _sc_penalty_body(log_hbm, fidx_hbm, out_hbm, idx_v, vals_v, sem):
    wid = lax.axis_index("s") * _NC + lax.axis_index("c")
    base = wid * _ELEMS_PER_W
    pltpu.sync_copy(log_hbm.at[pl.ds(base, _ELEMS_PER_W)],
                    out_hbm.at[pl.ds(base, _ELEMS_PER_W)])
    pltpu.sync_copy(fidx_hbm.at[wid], idx_v)
    descs = [pltpu.async_copy(log_hbm.at[idx_v.at[j]], vals_v.at[j], sem)
             for j in range(_IDX_ROWS)]
    for d in descs:
        d.wait()
    for j in range(_IDX_ROWS):
        for i in range(128 // _L):
            v = vals_v[j, pl.ds(i * _L, _L)]
            vals_v[j, pl.ds(i * _L, _L)] = jnp.where(
                v > 0.0, v / _REP_PENALTY, v * _REP_PENALTY)
    descs = [pltpu.async_copy(vals_v.at[j], out_hbm.at[idx_v.at[j]], sem)
             for j in range(_IDX_ROWS)]
    for d in descs:
        d.wait()


@functools.cache
def _sc_penalty():
    return pl.kernel(
        _sc_penalty_body,
        mesh=plsc.VectorSubcoreMesh(
            core_axis_name="c", subcore_axis_name="s", num_cores=_NC),
        out_type=jax.ShapeDtypeStruct((_B * _V,), jnp.float32),
        scratch_types=[
            pltpu.VMEM((_IDX_ROWS, 128), jnp.int32),
            pltpu.VMEM((_IDX_ROWS, 128), jnp.float32),
            pltpu.SemaphoreType.DMA,
        ],
    )


_ROWS_PER_BLK = 8
_GRID = _B // _ROWS_PER_BLK


def _avg_ceil(lo, hi):
    fl = (lo >> 1) + (hi >> 1) + (lo & hi & 1)
    return fl + ((lo ^ hi) & 1)


def _tc_body(kk_ref, x_ref, g_ref, probs_ref, tok_ref, s_ref, e_ref):
    x = x_ref[...] / jnp.float32(_TEMPERATURE)
    b = lax.bitcast_convert_type(x, jnp.int32)
    s = b ^ ((b >> 31) & jnp.int32(0x7FFFFFFF))
    s_ref[...] = s
    kk = kk_ref[0]

    nfull = _V // 128
    cm = s[:, 0:128]
    for i in range(1, nfull):
        cm = jnp.maximum(cm, s[:, 128 * i:128 * (i + 1)])
    ntail = _V - nfull * 128
    if ntail:
        head = jnp.maximum(cm[:, 0:ntail], s[:, nfull * 128:_V])
        cm = jnp.concatenate([head, cm[:, ntail:128]], axis=1)
    smax = jnp.max(cm, axis=1, keepdims=True)

    def cond1(c):
        return jnp.any(c[0] < c[1])

    kcm = jnp.minimum(kk, jnp.int32(128))

    def bodycm(c):
        lo, hi = c
        mid = _avg_ceil(lo, hi)
        cnt = jnp.sum((cm >= mid).astype(jnp.int32), axis=1, keepdims=True)
        ge = cnt >= kcm
        return (jnp.where(ge, mid, lo), jnp.where(ge, hi, mid - 1))

    cmin = jnp.min(cm, axis=1, keepdims=True)
    lkey, _ = lax.while_loop(cond1, bodycm, (cmin, smax))
    min_key = jnp.full_like(lkey, jnp.int32(-2139095041))
    lo0 = jnp.where(kk <= 128, lkey, min_key)

    def body1(c):
        lo, hi = c
        mid = _avg_ceil(lo, hi)
        cnt = jnp.sum((s_ref[...] >= mid).astype(jnp.int32), axis=1,
                      keepdims=True)
        ge = cnt >= kk
        return (jnp.where(ge, mid, lo), jnp.where(ge, hi, mid - 1))

    tstar, _ = lax.while_loop(cond1, body1, (lo0, smax))

    m = jnp.max(x, axis=1, keepdims=True)
    e = jnp.where(s >= tstar, jnp.exp(x - m), jnp.float32(0.0))
    e_ref[...] = e
    z = jnp.sum(e, axis=1, keepdims=True)
    target = z * jnp.float32(_TOP_P)

    def body2(c):
        lo, hi = c
        mid = _avg_ceil(lo, hi)
        mass = jnp.sum(jnp.where(s_ref[...] >= mid, e_ref[...],
                                 jnp.float32(0.0)), axis=1, keepdims=True)
        ge = mass >= target
        return (jnp.where(ge, mid, lo), jnp.where(ge, hi, mid - 1))

    t0, _ = lax.while_loop(cond1, body2, (tstar, smax))

    at_t0 = s_ref[...] == t0
    tie_cnt = jnp.sum(at_t0.astype(jnp.int32), axis=1, keepdims=True)
    mass_above = jnp.sum(jnp.where(s_ref[...] > t0, e_ref[...],
                                   jnp.float32(0.0)), axis=1, keepdims=True)
    e_tie = jnp.max(jnp.where(at_t0, e_ref[...], jnp.float32(0.0)),
                    axis=1, keepdims=True)
    n_keep = jnp.ceil((target - mass_above) / e_tie).astype(jnp.int32)

    def body3(c):
        lo, hi = c
        mid = (lo >> 1) + (hi >> 1) + (lo & hi & 1)
        ii = lax.broadcasted_iota(jnp.int32, at_t0.shape, 1)
        cnt = jnp.sum((at_t0 & (ii < mid)).astype(jnp.int32), axis=1,
                      keepdims=True)
        ge = cnt >= n_keep
        return (jnp.where(ge, lo, mid + 1), jnp.where(ge, mid, hi))

    vfull = jnp.zeros_like(t0) + jnp.int32(_V)
    lo3 = jnp.where(n_keep == tie_cnt, vfull, jnp.zeros_like(t0))
    cidx, _ = lax.while_loop(cond1, body3, (lo3, vfull))

    ii = lax.broadcasted_iota(jnp.int32, at_t0.shape, 1)
    keep = (s > t0) | (at_t0 & (ii < cidx))
    ekeep = jnp.where(keep, e, jnp.float32(0.0))
    zs = jnp.sum(ekeep, axis=1, keepdims=True)
    p = ekeep / zs
    probs_ref[...] = p

    score = jnp.log(p + jnp.float32(1e-20)) + g_ref[...]
    mx = jnp.max(score, axis=1, keepdims=True)
    ii = lax.broadcasted_iota(jnp.int32, score.shape, 1)
    idx = jnp.min(jnp.where(score >= mx, ii, jnp.int32(_V)), axis=1)
    tok_ref[...] = idx[:, None]


def _tc_sample(plog, gumbel, kk):
    grid_spec = pltpu.PrefetchScalarGridSpec(
        num_scalar_prefetch=1,
        grid=(_GRID,),
        in_specs=[
            pl.BlockSpec((_ROWS_PER_BLK, _V), lambda i, kk: (i, 0)),
            pl.BlockSpec((_ROWS_PER_BLK, _V), lambda i, kk: (i, 0)),
        ],
        out_specs=[
            pl.BlockSpec((_ROWS_PER_BLK, _V), lambda i, kk: (i, 0)),
            pl.BlockSpec((_ROWS_PER_BLK, 1), lambda i, kk: (i, 0)),
        ],
        scratch_shapes=[
            pltpu.VMEM((_ROWS_PER_BLK, _V), jnp.int32),
            pltpu.VMEM((_ROWS_PER_BLK, _V), jnp.float32),
        ],
    )
    return pl.pallas_call(
        _tc_body,
        grid_spec=grid_spec,
        out_shape=[
            jax.ShapeDtypeStruct((_B, _V), jnp.float32),
            jax.ShapeDtypeStruct((_B, 1), jnp.int32),
        ],
        compiler_params=pltpu.CompilerParams(
            dimension_semantics=("parallel",)),
    )(kk, plog, gumbel)


def kernel(logits, generated, top_k):
    fidx = (generated.astype(jnp.int32)
            + jnp.arange(_B, dtype=jnp.int32)[:, None] * _V)
    fidx = fidx.reshape(_NW, _IDX_ROWS, 128)
    plog = _sc_penalty()(logits.reshape(-1), fidx).reshape(_B, _V)
    kk = jnp.minimum(jnp.asarray(top_k, jnp.int32), _V).reshape(1)
    probs, tok = _tc_sample(plog, _gumbel(), kk)
    return probs, tok.reshape(_B)


# speedup vs baseline: 26.7740x; 1.1020x over previous
import functools

import jax
import jax.numpy as jnp
from jax import lax
from jax.experimental import pallas as pl
from jax.experimental.pallas import tpu as pltpu
from jax.experimental.pallas import tpu_sc as plsc

_B, _V = 128, 100000
_TEMPERATURE = 0.8
_TOP_P = 0.9
_REP_PENALTY = 1.2

def _gumbel():
    return jax.random.gumbel(jax.random.key(42), (_B, _V), dtype=jnp.float32)


_NC, _NS, _L = 2, 16, 16
_NW = _NC * _NS
_ROWS_PER_W = _B // _NW
_ELEMS_PER_W = _ROWS_PER_W * _V
_IDX_PER_W = _ROWS_PER_W * 512
_IDX_ROWS = _IDX_PER_W // 128


def _sc_penalty_body(log_hbm, fidx_hbm, out_hbm, idx_v, vals_v, sem):
    wid = lax.axis_index("s") * _NC + lax.axis_index("c")
    base = wid * _ELEMS_PER_W
    pltpu.sync_copy(log_hbm.at[pl.ds(base, _ELEMS_PER_W)],
                    out_hbm.at[pl.ds(base, _ELEMS_PER_W)])
    pltpu.sync_copy(fidx_hbm.at[wid], idx_v)
    descs = [pltpu.async_copy(log_hbm.at[idx_v.at[j]], vals_v.at[j], sem)
             for j in range(_IDX_ROWS)]
    for d in descs:
        d.wait()
    for j in range(_IDX_ROWS):
        for i in range(128 // _L):
            v = vals_v[j, pl.ds(i * _L, _L)]
            vals_v[j, pl.ds(i * _L, _L)] = jnp.where(
                v > 0.0, v / _REP_PENALTY, v * _REP_PENALTY)
    descs = [pltpu.async_copy(vals_v.at[j], out_hbm.at[idx_v.at[j]], sem)
             for j in range(_IDX_ROWS)]
    for d in descs:
        d.wait()


@functools.cache
def _sc_penalty():
    return pl.kernel(
        _sc_penalty_body,
        mesh=plsc.VectorSubcoreMesh(
            core_axis_name="c", subcore_axis_name="s", num_cores=_NC),
        out_type=jax.ShapeDtypeStruct((_B * _V,), jnp.float32),
        scratch_types=[
            pltpu.VMEM((_IDX_ROWS, 128), jnp.int32),
            pltpu.VMEM((_IDX_ROWS, 128), jnp.float32),
            pltpu.SemaphoreType.DMA,
        ],
    )


_ROWS_PER_BLK = 8
_GRID = _B // _ROWS_PER_BLK


def _avg_ceil(lo, hi):
    fl = (lo >> 1) + (hi >> 1) + (lo & hi & 1)
    return fl + ((lo ^ hi) & 1)


def _tc_body(kk_ref, x_ref, g_ref, probs_ref, tok_ref, s_ref, e_ref):
    x = x_ref[...] / jnp.float32(_TEMPERATURE)
    b = lax.bitcast_convert_type(x, jnp.int32)
    s = b ^ ((b >> 31) & jnp.int32(0x7FFFFFFF))
    s_ref[...] = s
    kk = kk_ref[0]

    nfull = _V // 128
    cm = s[:, 0:128]
    for i in range(1, nfull):
        cm = jnp.maximum(cm, s[:, 128 * i:128 * (i + 1)])
    ntail = _V - nfull * 128
    if ntail:
        head = jnp.maximum(cm[:, 0:ntail], s[:, nfull * 128:_V])
        cm = jnp.concatenate([head, cm[:, ntail:128]], axis=1)
    smax = jnp.max(cm, axis=1, keepdims=True)

    def cond1(c):
        return jnp.any(c[0] < c[1])

    kcm = jnp.minimum(kk, jnp.int32(128))

    def bodycm(c):
        lo, hi = c
        mid = _avg_ceil(lo, hi)
        cnt = jnp.sum((cm >= mid).astype(jnp.int32), axis=1, keepdims=True)
        ge = cnt >= kcm
        return (jnp.where(ge, mid, lo), jnp.where(ge, hi, mid - 1))

    cmin = jnp.min(cm, axis=1, keepdims=True)
    lkey, _ = lax.while_loop(cond1, bodycm, (cmin, smax))
    min_key = jnp.full_like(lkey, jnp.int32(-2139095041))
    lo0 = jnp.where(kk <= 128, lkey, min_key)

    def body1(c):
        lo, hi = c
        sv = s_ref[...]
        mid = _avg_ceil(lo, hi)
        ma = _avg_ceil(mid, hi)
        mb = _avg_ceil(lo, mid - 1)
        c0 = jnp.sum((sv >= mid).astype(jnp.int32), axis=1, keepdims=True)
        ca = jnp.sum((sv >= ma).astype(jnp.int32), axis=1, keepdims=True)
        cb = jnp.sum((sv >= mb).astype(jnp.int32), axis=1, keepdims=True)
        ge, gea, geb = c0 >= kk, ca >= kk, cb >= kk
        lo2 = jnp.where(ge, jnp.where(gea, ma, mid), jnp.where(geb, mb, lo))
        hi2 = jnp.where(ge, jnp.where(gea, hi, ma - 1),
                        jnp.where(geb, mid - 1, mb - 1))
        return (lo2, hi2)

    tstar, _ = lax.while_loop(cond1, body1, (lo0, smax))

    m = jnp.max(x, axis=1, keepdims=True)
    e = jnp.where(s >= tstar, jnp.exp(x - m), jnp.float32(0.0))
    e_ref[...] = e
    z = jnp.sum(e, axis=1, keepdims=True)
    target = z * jnp.float32(_TOP_P)

    def body2(c):
        lo, hi = c
        sv = s_ref[...]
        ev = e_ref[...]
        zf = jnp.float32(0.0)
        mid = _avg_ceil(lo, hi)
        ma = _avg_ceil(mid, hi)
        mb = _avg_ceil(lo, mid - 1)
        g0 = jnp.sum(jnp.where(sv >= mid, ev, zf), axis=1, keepdims=True)
        ga = jnp.sum(jnp.where(sv >= ma, ev, zf), axis=1, keepdims=True)
        gb = jnp.sum(jnp.where(sv >= mb, ev, zf), axis=1, keepdims=True)
        ge, gea, geb = g0 >= target, ga >= target, gb >= target
        lo2 = jnp.where(ge, jnp.where(gea, ma, mid), jnp.where(geb, mb, lo))
        hi2 = jnp.where(ge, jnp.where(gea, hi, ma - 1),
                        jnp.where(geb, mid - 1, mb - 1))
        return (lo2, hi2)

    t0, _ = lax.while_loop(cond1, body2, (tstar, smax))

    at_t0 = s_ref[...] == t0
    tie_cnt = jnp.sum(at_t0.astype(jnp.int32), axis=1, keepdims=True)
    mass_above = jnp.sum(jnp.where(s_ref[...] > t0, e_ref[...],
                                   jnp.float32(0.0)), axis=1, keepdims=True)
    e_tie = jnp.max(jnp.where(at_t0, e_ref[...], jnp.float32(0.0)),
                    axis=1, keepdims=True)
    n_keep = jnp.ceil((target - mass_above) / e_tie).astype(jnp.int32)

    def body3(c):
        lo, hi = c
        mid = (lo >> 1) + (hi >> 1) + (lo & hi & 1)
        ii = lax.broadcasted_iota(jnp.int32, at_t0.shape, 1)
        cnt = jnp.sum((at_t0 & (ii < mid)).astype(jnp.int32), axis=1,
                      keepdims=True)
        ge = cnt >= n_keep
        return (jnp.where(ge, lo, mid + 1), jnp.where(ge, mid, hi))

    vfull = jnp.zeros_like(t0) + jnp.int32(_V)
    lo3 = jnp.where(n_keep == tie_cnt, vfull, jnp.zeros_like(t0))
    cidx, _ = lax.while_loop(cond1, body3, (lo3, vfull))

    ii = lax.broadcasted_iota(jnp.int32, at_t0.shape, 1)
    keep = (s > t0) | (at_t0 & (ii < cidx))
    ekeep = jnp.where(keep, e, jnp.float32(0.0))
    zs = jnp.sum(ekeep, axis=1, keepdims=True)
    p = ekeep / zs
    probs_ref[...] = p

    score = jnp.log(p + jnp.float32(1e-20)) + g_ref[...]
    mx = jnp.max(score, axis=1, keepdims=True)
    ii = lax.broadcasted_iota(jnp.int32, score.shape, 1)
    idx = jnp.min(jnp.where(score >= mx, ii, jnp.int32(_V)), axis=1)
    tok_ref[...] = idx[:, None]


def _tc_sample(plog, gumbel, kk):
    grid_spec = pltpu.PrefetchScalarGridSpec(
        num_scalar_prefetch=1,
        grid=(_GRID,),
        in_specs=[
            pl.BlockSpec((_ROWS_PER_BLK, _V), lambda i, kk: (i, 0)),
            pl.BlockSpec((_ROWS_PER_BLK, _V), lambda i, kk: (i, 0)),
        ],
        out_specs=[
            pl.BlockSpec((_ROWS_PER_BLK, _V), lambda i, kk: (i, 0)),
            pl.BlockSpec((_ROWS_PER_BLK, 1), lambda i, kk: (i, 0)),
        ],
        scratch_shapes=[
            pltpu.VMEM((_ROWS_PER_BLK, _V), jnp.int32),
            pltpu.VMEM((_ROWS_PER_BLK, _V), jnp.float32),
        ],
    )
    return pl.pallas_call(
        _tc_body,
        grid_spec=grid_spec,
        out_shape=[
            jax.ShapeDtypeStruct((_B, _V), jnp.float32),
            jax.ShapeDtypeStruct((_B, 1), jnp.int32),
        ],
        compiler_params=pltpu.CompilerParams(
            dimension_semantics=("parallel",)),
    )(kk, plog, gumbel)


def kernel(logits, generated, top_k):
    fidx = (generated.astype(jnp.int32)
            + jnp.arange(_B, dtype=jnp.int32)[:, None] * _V)
    fidx = fidx.reshape(_NW, _IDX_ROWS, 128)
    plog = _sc_penalty()(logits.reshape(-1), fidx).reshape(_B, _V)
    kk = jnp.minimum(jnp.asarray(top_k, jnp.int32), _V).reshape(1)
    probs, tok = _tc_sample(plog, _gumbel(), kk)
    return probs, tok.reshape(_B)


# speedup vs baseline: 26.8123x; 1.0014x over previous
import functools

import jax
import jax.numpy as jnp
from jax import lax
from jax.experimental import pallas as pl
from jax.experimental.pallas import tpu as pltpu
from jax.experimental.pallas import tpu_sc as plsc

_B, _V = 128, 100000
_TEMPERATURE = 0.8
_TOP_P = 0.9
_REP_PENALTY = 1.2

def _gumbel():
    return jax.random.gumbel(jax.random.key(42), (_B, _V), dtype=jnp.float32)


_NC, _NS, _L = 2, 16, 16
_NW = _NC * _NS
_ROWS_PER_W = _B // _NW
_ELEMS_PER_W = _ROWS_PER_W * _V
_IDX_PER_W = _ROWS_PER_W * 512
_IDX_ROWS = _IDX_PER_W // 128


_COPY_CHUNKS = 1
_CHUNK = _ELEMS_PER_W // _COPY_CHUNKS


def _sc_penalty_body(log_hbm, fidx_hbm, out_hbm, idx_v, vals_v, sem, csem):
    wid = lax.axis_index("s") * _NC + lax.axis_index("c")
    base = wid * _ELEMS_PER_W
    cds = [pltpu.async_copy(log_hbm.at[pl.ds(base + i * _CHUNK, _CHUNK)],
                            out_hbm.at[pl.ds(base + i * _CHUNK, _CHUNK)],
                            csem)
           for i in range(_COPY_CHUNKS)]
    pltpu.sync_copy(fidx_hbm.at[wid], idx_v)
    descs = [pltpu.async_copy(log_hbm.at[idx_v.at[j]], vals_v.at[j], sem)
             for j in range(_IDX_ROWS)]
    for d in descs:
        d.wait()
    for j in range(_IDX_ROWS):
        for i in range(128 // _L):
            v = vals_v[j, pl.ds(i * _L, _L)]
            vals_v[j, pl.ds(i * _L, _L)] = jnp.where(
                v > 0.0, v / _REP_PENALTY, v * _REP_PENALTY)
    for d in cds:
        d.wait()
    descs = [pltpu.async_copy(vals_v.at[j], out_hbm.at[idx_v.at[j]], sem)
             for j in range(_IDX_ROWS)]
    for d in descs:
        d.wait()


@functools.cache
def _sc_penalty():
    return pl.kernel(
        _sc_penalty_body,
        mesh=plsc.VectorSubcoreMesh(
            core_axis_name="c", subcore_axis_name="s", num_cores=_NC),
        out_type=jax.ShapeDtypeStruct((_B * _V,), jnp.float32),
        scratch_types=[
            pltpu.VMEM((_IDX_ROWS, 128), jnp.int32),
            pltpu.VMEM((_IDX_ROWS, 128), jnp.float32),
            pltpu.SemaphoreType.DMA,
            pltpu.SemaphoreType.DMA,
        ],
    )


_ROWS_PER_BLK = 8
_GRID = _B // _ROWS_PER_BLK


def _avg_ceil(lo, hi):
    fl = (lo >> 1) + (hi >> 1) + (lo & hi & 1)
    return fl + ((lo ^ hi) & 1)


def _tc_body(kk_ref, x_ref, g_ref, probs_ref, tok_ref, s_ref, e_ref):
    x = x_ref[...] / jnp.float32(_TEMPERATURE)
    b = lax.bitcast_convert_type(x, jnp.int32)
    s = b ^ ((b >> 31) & jnp.int32(0x7FFFFFFF))
    s_ref[...] = s
    kk = kk_ref[0]

    nfull = _V // 128
    cm = s[:, 0:128]
    for i in range(1, nfull):
        cm = jnp.maximum(cm, s[:, 128 * i:128 * (i + 1)])
    ntail = _V - nfull * 128
    if ntail:
        head = jnp.maximum(cm[:, 0:ntail], s[:, nfull * 128:_V])
        cm = jnp.concatenate([head, cm[:, ntail:128]], axis=1)
    smax = jnp.max(cm, axis=1, keepdims=True)

    def cond1(c):
        return jnp.any(c[0] < c[1])

    kcm = jnp.minimum(kk, jnp.int32(128))

    def bodycm(c):
        lo, hi = c
        mid = _avg_ceil(lo, hi)
        cnt = jnp.sum((cm >= mid).astype(jnp.int32), axis=1, keepdims=True)
        ge = cnt >= kcm
        return (jnp.where(ge, mid, lo), jnp.where(ge, hi, mid - 1))

    cmin = jnp.min(cm, axis=1, keepdims=True)
    lkey, _ = lax.while_loop(cond1, bodycm, (cmin, smax))
    min_key = jnp.full_like(lkey, jnp.int32(-2139095041))
    lo0 = jnp.where(kk <= 128, lkey, min_key)

    def body1(c):
        lo, hi = c
        sv = s_ref[...]
        mid = _avg_ceil(lo, hi)
        ma = _avg_ceil(mid, hi)
        mb = _avg_ceil(lo, mid - 1)
        c0 = jnp.sum((sv >= mid).astype(jnp.int32), axis=1, keepdims=True)
        ca = jnp.sum((sv >= ma).astype(jnp.int32), axis=1, keepdims=True)
        cb = jnp.sum((sv >= mb).astype(jnp.int32), axis=1, keepdims=True)
        ge, gea, geb = c0 >= kk, ca >= kk, cb >= kk
        lo2 = jnp.where(ge, jnp.where(gea, ma, mid), jnp.where(geb, mb, lo))
        hi2 = jnp.where(ge, jnp.where(gea, hi, ma - 1),
                        jnp.where(geb, mid - 1, mb - 1))
        return (lo2, hi2)

    tstar, _ = lax.while_loop(cond1, body1, (lo0, smax))

    m = jnp.max(x, axis=1, keepdims=True)
    e = jnp.where(s >= tstar, jnp.exp(x - m), jnp.float32(0.0))
    e_ref[...] = e
    z = jnp.sum(e, axis=1, keepdims=True)
    target = z * jnp.float32(_TOP_P)

    def body2(c):
        lo, hi = c
        sv = s_ref[...]
        ev = e_ref[...]
        zf = jnp.float32(0.0)
        mid = _avg_ceil(lo, hi)
        ma = _avg_ceil(mid, hi)
        mb = _avg_ceil(lo, mid - 1)
        g0 = jnp.sum(jnp.where(sv >= mid, ev, zf), axis=1, keepdims=True)
        ga = jnp.sum(jnp.where(sv >= ma, ev, zf), axis=1, keepdims=True)
        gb = jnp.sum(jnp.where(sv >= mb, ev, zf), axis=1, keepdims=True)
        ge, gea, geb = g0 >= target, ga >= target, gb >= target
        lo2 = jnp.where(ge, jnp.where(gea, ma, mid), jnp.where(geb, mb, lo))
        hi2 = jnp.where(ge, jnp.where(gea, hi, ma - 1),
                        jnp.where(geb, mid - 1, mb - 1))
        return (lo2, hi2)

    t0, _ = lax.while_loop(cond1, body2, (tstar, smax))

    at_t0 = s_ref[...] == t0
    tie_cnt = jnp.sum(at_t0.astype(jnp.int32), axis=1, keepdims=True)
    mass_above = jnp.sum(jnp.where(s_ref[...] > t0, e_ref[...],
                                   jnp.float32(0.0)), axis=1, keepdims=True)
    e_tie = jnp.max(jnp.where(at_t0, e_ref[...], jnp.float32(0.0)),
                    axis=1, keepdims=True)
    n_keep = jnp.ceil((target - mass_above) / e_tie).astype(jnp.int32)

    def body3(c):
        lo, hi = c
        mid = (lo >> 1) + (hi >> 1) + (lo & hi & 1)
        ii = lax.broadcasted_iota(jnp.int32, at_t0.shape, 1)
        cnt = jnp.sum((at_t0 & (ii < mid)).astype(jnp.int32), axis=1,
                      keepdims=True)
        ge = cnt >= n_keep
        return (jnp.where(ge, lo, mid + 1), jnp.where(ge, mid, hi))

    vfull = jnp.zeros_like(t0) + jnp.int32(_V)
    lo3 = jnp.where(n_keep == tie_cnt, vfull, jnp.zeros_like(t0))
    cidx, _ = lax.while_loop(cond1, body3, (lo3, vfull))

    ii = lax.broadcasted_iota(jnp.int32, at_t0.shape, 1)
    keep = (s > t0) | (at_t0 & (ii < cidx))
    ekeep = jnp.where(keep, e, jnp.float32(0.0))
    zs = jnp.sum(ekeep, axis=1, keepdims=True)
    p = ekeep / zs
    probs_ref[...] = p

    score = jnp.log(p + jnp.float32(1e-20)) + g_ref[...]
    mx = jnp.max(score, axis=1, keepdims=True)
    ii = lax.broadcasted_iota(jnp.int32, score.shape, 1)
    idx = jnp.min(jnp.where(score >= mx, ii, jnp.int32(_V)), axis=1)
    tok_ref[...] = idx[:, None]


def _tc_sample(plog, gumbel, kk):
    grid_spec = pltpu.PrefetchScalarGridSpec(
        num_scalar_prefetch=1,
        grid=(_GRID,),
        in_specs=[
            pl.BlockSpec((_ROWS_PER_BLK, _V), lambda i, kk: (i, 0)),
            pl.BlockSpec((_ROWS_PER_BLK, _V), lambda i, kk: (i, 0)),
        ],
        out_specs=[
            pl.BlockSpec((_ROWS_PER_BLK, _V), lambda i, kk: (i, 0)),
            pl.BlockSpec((_ROWS_PER_BLK, 1), lambda i, kk: (i, 0)),
        ],
        scratch_shapes=[
            pltpu.VMEM((_ROWS_PER_BLK, _V), jnp.int32),
            pltpu.VMEM((_ROWS_PER_BLK, _V), jnp.float32),
        ],
    )
    return pl.pallas_call(
        _tc_body,
        grid_spec=grid_spec,
        out_shape=[
            jax.ShapeDtypeStruct((_B, _V), jnp.float32),
            jax.ShapeDtypeStruct((_B, 1), jnp.int32),
        ],
        compiler_params=pltpu.CompilerParams(
            dimension_semantics=("parallel",)),
    )(kk, plog, gumbel)


def kernel(logits, generated, top_k):
    fidx = (generated.astype(jnp.int32)
            + jnp.arange(_B, dtype=jnp.int32)[:, None] * _V)
    fidx = fidx.reshape(_NW, _IDX_ROWS, 128)
    plog = _sc_penalty()(logits.reshape(-1), fidx).reshape(_B, _V)
    kk = jnp.minimum(jnp.asarray(top_k, jnp.int32), _V).reshape(1)
    probs, tok = _tc_sample(plog, _gumbel(), kk)
    return probs, tok.reshape(_B)
